# baseline (speedup 1.0000x reference)
import functools

import jax
import jax.numpy as jnp
from jax import lax
from jax.experimental import pallas as pl
from jax.experimental.pallas import tpu as pltpu
from jax.experimental.pallas import tpu_sc as plsc

VOCAB = 1000
BATCH = 1024
SEQ = 200
N_ROWS = BATCH * SEQ
NUM_WORKERS = 32
ROWS_PER_W = N_ROWS // NUM_WORKERS
CHUNK = 64
N_CHUNKS = ROWS_PER_W // CHUNK


def _emb_body(x_hbm, table_hbm, out_hbm, idx_v, rows_v, sem):
    wid = lax.axis_index("s") * 2 + lax.axis_index("c")
    base = wid * ROWS_PER_W
    pltpu.sync_copy(x_hbm.at[wid], idx_v)

    def body(j, carry):
        pltpu.async_copy(table_hbm.at[idx_v.at[j]], rows_v, sem).wait()
        pltpu.sync_copy(rows_v, out_hbm.at[pl.ds(base + j * CHUNK, CHUNK)])
        return carry

    lax.fori_loop(0, N_CHUNKS, body, 0)


@functools.partial(jax.jit, static_argnums=())
def _emb_call(xw, table):
    mesh = plsc.VectorSubcoreMesh(core_axis_name="c", subcore_axis_name="s")
    f = functools.partial(
        pl.kernel,
        mesh=mesh,
        out_type=jax.ShapeDtypeStruct((N_ROWS, VOCAB), jnp.float32),
        scratch_types=[
            pltpu.VMEM((N_CHUNKS, CHUNK), jnp.int32),
            pltpu.VMEM((CHUNK, VOCAB), jnp.float32),
            pltpu.SemaphoreType.DMA,
        ],
        compiler_params=pltpu.CompilerParams(use_tc_tiling_on_sc=False),
    )(_emb_body)
    return f(xw, table)


def kernel(x, table):
    xw = x.reshape(NUM_WORKERS, N_CHUNKS, CHUNK).astype(jnp.int32)
    out = _emb_call(xw, table)
    return out.reshape(BATCH, SEQ, VOCAB)


# speedup vs baseline: 1.0185x; 1.0185x over previous
import functools

import jax
import jax.numpy as jnp
from jax import lax
from jax.experimental import pallas as pl
from jax.experimental.pallas import tpu as pltpu
from jax.experimental.pallas import tpu_sc as plsc

VOCAB = 1000
BATCH = 1024
SEQ = 200
N_ROWS = BATCH * SEQ
NUM_WORKERS = 32
ROWS_PER_W = N_ROWS // NUM_WORKERS
CHUNK = 50
N_CHUNKS = ROWS_PER_W // CHUNK
N_PAIRS = N_CHUNKS // 2


def _emb_body(x_hbm, table_hbm, out_hbm, idx_v, rows_a, rows_b, gsem, wsem):
    wid = lax.axis_index("s") * 2 + lax.axis_index("c")
    base = wid * ROWS_PER_W
    pltpu.sync_copy(x_hbm.at[wid], idx_v)

    pltpu.async_copy(table_hbm.at[idx_v.at[0]], rows_a, gsem)

    def body(t, carry):
        j0 = 2 * t
        gb = pltpu.async_copy(table_hbm.at[idx_v.at[j0 + 1]], rows_b, gsem)
        pltpu.make_async_copy(table_hbm.at[pl.ds(0, CHUNK)], rows_a, gsem).wait()
        wa = pltpu.async_copy(rows_a, out_hbm.at[pl.ds(base + j0 * CHUNK, CHUNK)], wsem)
        gb.wait()
        wb = pltpu.async_copy(rows_b, out_hbm.at[pl.ds(base + (j0 + 1) * CHUNK, CHUNK)], wsem)
        wa.wait()

        @pl.when(t + 1 < N_PAIRS)
        def _():
            pltpu.async_copy(table_hbm.at[idx_v.at[j0 + 2]], rows_a, gsem)

        wb.wait()
        return carry

    lax.fori_loop(0, N_PAIRS, body, 0)


@functools.partial(jax.jit, static_argnums=())
def _emb_call(xw, table):
    mesh = plsc.VectorSubcoreMesh(core_axis_name="c", subcore_axis_name="s")
    f = functools.partial(
        pl.kernel,
        mesh=mesh,
        out_type=jax.ShapeDtypeStruct((N_ROWS, VOCAB), jnp.float32),
        scratch_types=[
            pltpu.VMEM((N_CHUNKS, CHUNK), jnp.int32),
            pltpu.VMEM((CHUNK, VOCAB), jnp.float32),
            pltpu.VMEM((CHUNK, VOCAB), jnp.float32),
            pltpu.SemaphoreType.DMA,
            pltpu.SemaphoreType.DMA,
        ],
        compiler_params=pltpu.CompilerParams(use_tc_tiling_on_sc=False),
    )(_emb_body)
    return f(xw, table)


def kernel(x, table):
    xw = x.reshape(NUM_WORKERS, N_CHUNKS, CHUNK).astype(jnp.int32)
    out = _emb_call(xw, table)
    return out.reshape(BATCH, SEQ, VOCAB)
